# Initial kernel scaffold; baseline (speedup 1.0000x reference)
import functools

import jax
import jax.numpy as jnp
from jax import lax
from jax.experimental import pallas as pl
from jax.experimental.pallas import tpu as pltpu
from jax.experimental.pallas import tpu_sc as plsc

NC = 2
NS = 16
LANES = 16
NW = NC * NS
K = 128


def _full16(v):
    return jnp.full((LANES,), v, dtype=jnp.int32)


def _mm1_body(x_ref, w_ref, b_ref, o_ref):
    o_ref[...] = lax.dot_general(
        x_ref[...], w_ref[...], (((1,), (1,)), ((), ())),
        preferred_element_type=jnp.float32,
    ) + b_ref[...]


def _mm2_body(p0_ref, p1_ref, w_ref, b_ref, o_ref):
    h = jnp.maximum(p0_ref[...] + p1_ref[...], 0.0)
    o_ref[...] = lax.dot_general(
        h, w_ref[...], (((1,), (1,)), ((), ())),
        preferred_element_type=jnp.float32,
    ) + b_ref[...]


def _add2_body(q0_ref, q1_ref, o_ref):
    o_ref[...] = q0_ref[...] + q1_ref[...]


def _mm1(x, w, b, bn):
    n, d = x.shape
    return pl.pallas_call(
        _mm1_body,
        grid=(n // bn,),
        in_specs=[
            pl.BlockSpec((bn, d), lambda i: (i, 0)),
            pl.BlockSpec(w.shape, lambda i: (0, 0)),
            pl.BlockSpec((1, d), lambda i: (0, 0)),
        ],
        out_specs=pl.BlockSpec((bn, w.shape[0]), lambda i: (i, 0)),
        out_shape=jax.ShapeDtypeStruct((n, w.shape[0]), jnp.float32),
    )(x, w, b.reshape(1, -1))


def _mm2(p0, p1, w, b, bn):
    n, d = p0.shape
    return pl.pallas_call(
        _mm2_body,
        grid=(n // bn,),
        in_specs=[
            pl.BlockSpec((bn, d), lambda i: (i, 0)),
            pl.BlockSpec((bn, d), lambda i: (i, 0)),
            pl.BlockSpec(w.shape, lambda i: (0, 0)),
            pl.BlockSpec((1, d), lambda i: (0, 0)),
        ],
        out_specs=pl.BlockSpec((bn, w.shape[0]), lambda i: (i, 0)),
        out_shape=jax.ShapeDtypeStruct((n, w.shape[0]), jnp.float32),
    )(p0, p1, w, b.reshape(1, -1))


def _add2(q0, q1, bn):
    n, d = q0.shape
    return pl.pallas_call(
        _add2_body,
        grid=(n // bn,),
        in_specs=[
            pl.BlockSpec((bn, d), lambda i: (i, 0)),
            pl.BlockSpec((bn, d), lambda i: (i, 0)),
        ],
        out_specs=pl.BlockSpec((bn, d), lambda i: (i, 0)),
        out_shape=jax.ShapeDtypeStruct((n, d), jnp.float32),
    )(q0, q1)


def _make_spmm(n, d, n_chunks):
    rpt = n // NS
    mesh = plsc.VectorSubcoreMesh(core_axis_name="c", subcore_axis_name="s")

    @functools.partial(
        pl.kernel,
        out_type=jax.ShapeDtypeStruct((NC, n, d), jnp.float32),
        mesh=mesh,
        scratch_types=[
            pltpu.VMEM((3, K), jnp.int32),
            pltpu.VMEM((K, d), jnp.float32),
            pltpu.VMEM_SHARED((n, d), jnp.float32),
            pltpu.SemaphoreType.DMA,
        ],
    )
    def spmm(h_hbm, epk_hbm, zero_hbm, out_hbm, ebuf, rows, acc, sem):
        c = lax.axis_index("c")
        s = lax.axis_index("s")
        wid = s * NC + c
        pltpu.sync_copy(zero_hbm.at[pl.ds(s * rpt, rpt)],
                        acc.at[pl.ds(s * rpt, rpt)])
        plsc.subcore_barrier()

        def chunk_body(i, carry):
            pltpu.sync_copy(epk_hbm.at[wid, i], ebuf)
            pltpu.async_copy(h_hbm.at[ebuf.at[0]], rows, sem).wait()

            def edge_body(e, ecarry):
                iv = plsc.load_gather(ebuf, [_full16(2), _full16(e)])
                val = plsc.bitcast(iv, jnp.float32)
                for j in range(d // LANES):
                    sl = (e, pl.ds(j * LANES, LANES))
                    rows[sl] = rows[sl] * val
                return ecarry

            lax.fori_loop(0, K, edge_body, 0)
            pltpu.sync_copy(rows, acc.at[ebuf.at[1]], add=True)
            return carry

        lax.fori_loop(0, n_chunks, chunk_body, 0)
        plsc.subcore_barrier()
        pltpu.sync_copy(acc.at[pl.ds(s * rpt, rpt)],
                        out_hbm.at[c, pl.ds(s * rpt, rpt)])

    return spmm


def kernel(x, edge_index, edge_values, W1, b1, W2, b2):
    n, d_in = x.shape
    e = edge_values.shape[0]
    n_chunks = -(-e // (NW * K))
    e_pad = NW * K * n_chunks
    pad = e_pad - e

    dst = edge_index[0].astype(jnp.int32)
    src = edge_index[1].astype(jnp.int32)
    vbits = lax.bitcast_convert_type(edge_values, jnp.int32)
    if pad:
        dst = jnp.pad(dst, (0, pad))
        src = jnp.pad(src, (0, pad))
        vbits = jnp.pad(vbits, (0, pad))
    epk = jnp.stack([src, dst, vbits]).reshape(3, NW, n_chunks, K)
    epk = jnp.transpose(epk, (1, 2, 0, 3))

    zeros = jnp.zeros((n, d_in), jnp.float32)
    spmm = _make_spmm(n, d_in, n_chunks)

    h1 = _mm1(x, W1, b1, 1000)
    p = spmm(h1, epk, zeros)
    h2 = _mm2(p[0], p[1], W2, b2, 1000)
    q = spmm(h2, epk, zeros)
    return _add2(q[0], q[1], 1000)


# speedup vs baseline: 3.6816x; 3.6816x over previous
import functools

import jax
import jax.numpy as jnp
from jax import lax
from jax.experimental import pallas as pl
from jax.experimental.pallas import tpu as pltpu
from jax.experimental.pallas import tpu_sc as plsc

NC = 2
NS = 16
LANES = 16
NW = NC * NS
K = 128


def _full16(v):
    return jnp.full((LANES,), v, dtype=jnp.int32)


_GATHER_DNUMS = lax.GatherDimensionNumbers(
    offset_dims=(), collapsed_slice_dims=(0,), start_index_map=(0,))


def _bcast_lane(vec, l):
    idx = jnp.full((LANES, 1), l, dtype=jnp.int32)
    return lax.gather(vec, idx, _GATHER_DNUMS, (1,),
                      mode=lax.GatherScatterMode.PROMISE_IN_BOUNDS)


def _mm1_body(x_ref, w_ref, b_ref, o_ref):
    o_ref[...] = lax.dot_general(
        x_ref[...], w_ref[...], (((1,), (1,)), ((), ())),
        preferred_element_type=jnp.float32,
    ) + b_ref[...]


def _mm2_body(p0_ref, p1_ref, w_ref, b_ref, o_ref):
    h = jnp.maximum(p0_ref[...] + p1_ref[...], 0.0)
    o_ref[...] = lax.dot_general(
        h, w_ref[...], (((1,), (1,)), ((), ())),
        preferred_element_type=jnp.float32,
    ) + b_ref[...]


def _add2_body(q0_ref, q1_ref, o_ref):
    o_ref[...] = q0_ref[...] + q1_ref[...]


def _mm1(x, w, b, bn):
    n, d = x.shape
    return pl.pallas_call(
        _mm1_body,
        grid=(n // bn,),
        in_specs=[
            pl.BlockSpec((bn, d), lambda i: (i, 0)),
            pl.BlockSpec(w.shape, lambda i: (0, 0)),
            pl.BlockSpec((1, d), lambda i: (0, 0)),
        ],
        out_specs=pl.BlockSpec((bn, w.shape[0]), lambda i: (i, 0)),
        out_shape=jax.ShapeDtypeStruct((n, w.shape[0]), jnp.float32),
    )(x, w, b.reshape(1, -1))


def _mm2(p0, p1, w, b, bn):
    n, d = p0.shape
    return pl.pallas_call(
        _mm2_body,
        grid=(n // bn,),
        in_specs=[
            pl.BlockSpec((bn, d), lambda i: (i, 0)),
            pl.BlockSpec((bn, d), lambda i: (i, 0)),
            pl.BlockSpec(w.shape, lambda i: (0, 0)),
            pl.BlockSpec((1, d), lambda i: (0, 0)),
        ],
        out_specs=pl.BlockSpec((bn, w.shape[0]), lambda i: (i, 0)),
        out_shape=jax.ShapeDtypeStruct((n, w.shape[0]), jnp.float32),
    )(p0, p1, w, b.reshape(1, -1))


def _add2(q0, q1, bn):
    n, d = q0.shape
    return pl.pallas_call(
        _add2_body,
        grid=(n // bn,),
        in_specs=[
            pl.BlockSpec((bn, d), lambda i: (i, 0)),
            pl.BlockSpec((bn, d), lambda i: (i, 0)),
        ],
        out_specs=pl.BlockSpec((bn, d), lambda i: (i, 0)),
        out_shape=jax.ShapeDtypeStruct((n, d), jnp.float32),
    )(q0, q1)


def _make_spmm(n_acc, d, n_chunks):
    rpt = n_acc // NS
    mesh = plsc.VectorSubcoreMesh(core_axis_name="c", subcore_axis_name="s")

    @functools.partial(
        pl.kernel,
        out_type=jax.ShapeDtypeStruct((NC, n_acc, d), jnp.float32),
        mesh=mesh,
        scratch_types=[
            pltpu.VMEM((2, K), jnp.int32),
            pltpu.VMEM((K,), jnp.float32),
            pltpu.VMEM((K, d), jnp.float32),
            pltpu.VMEM_SHARED((n_acc, d), jnp.float32),
            pltpu.SemaphoreType.DMA,
        ],
    )
    def spmm(h_hbm, epk_hbm, vals_hbm, zero_hbm, out_hbm,
             ebuf, vbuf, rows, acc, sem):
        c = lax.axis_index("c")
        s = lax.axis_index("s")
        wid = s * NC + c
        pltpu.sync_copy(zero_hbm.at[pl.ds(s * rpt, rpt)],
                        acc.at[pl.ds(s * rpt, rpt)])
        plsc.subcore_barrier()

        def chunk_body(i, carry):
            pltpu.sync_copy(epk_hbm.at[wid, i], ebuf)
            pltpu.sync_copy(vals_hbm.at[wid, i], vbuf)
            pltpu.async_copy(h_hbm.at[ebuf.at[0]], rows, sem).wait()

            def grp_body(g, gcarry):
                vv = vbuf[pl.ds(g * LANES, LANES)]
                for l in range(LANES):
                    val = _bcast_lane(vv, l)
                    e = g * LANES + l
                    for j in range(d // LANES):
                        sl = (e, pl.ds(j * LANES, LANES))
                        rows[sl] = rows[sl] * val
                return gcarry

            lax.fori_loop(0, K // LANES, grp_body, 0)
            pltpu.sync_copy(rows, acc.at[ebuf.at[1]], add=True)
            return carry

        lax.fori_loop(0, n_chunks, chunk_body, 0)
        plsc.subcore_barrier()
        pltpu.sync_copy(acc.at[pl.ds(s * rpt, rpt)],
                        out_hbm.at[c, pl.ds(s * rpt, rpt)])

    return spmm


def kernel(x, edge_index, edge_values, W1, b1, W2, b2):
    n, d_in = x.shape
    e = edge_values.shape[0]
    n_chunks = -(-e // (NW * K))
    e_pad = NW * K * n_chunks
    pad = e_pad - e

    dst = edge_index[0].astype(jnp.int32)
    src = edge_index[1].astype(jnp.int32)
    vals = edge_values
    if pad:
        dst = jnp.pad(dst, (0, pad))
        src = jnp.pad(src, (0, pad))
        vals = jnp.pad(vals, (0, pad))
    epk = jnp.stack([src, dst]).reshape(2, NW, n_chunks, K)
    epk = jnp.transpose(epk, (1, 2, 0, 3))
    valsp = vals.reshape(NW, n_chunks, K)

    n_acc = NS * (-(-n // (NS * 8)) * 8)
    zeros = jnp.zeros((n_acc, d_in), jnp.float32)
    spmm = _make_spmm(n_acc, d_in, n_chunks)

    h1 = _mm1(x, W1, b1, 1000)
    p = spmm(h1, epk, valsp, zeros)
    h2 = _mm2(p[0, :n], p[1, :n], W2, b2, 1000)
    q = spmm(h2, epk, valsp, zeros)
    return _add2(q[0, :n], q[1, :n], 1000)
